# Initial kernel scaffold; baseline (speedup 1.0000x reference)
import jax
import jax.numpy as jnp
from jax.experimental import pallas as pl


def kernel(inputs, class_embed, position_table):
    raise NotImplementedError("write your pallas kernel here")


# speedup vs baseline: 1.0079x; 1.0079x over previous
import jax
import jax.numpy as jnp
from jax.experimental import pallas as pl


def _body(in_ref, ce_ref, pe0_ref, pe_ref, out_ref):
    out_ref[0, 0:1, :] = ce_ref[0] + pe0_ref[...]
    out_ref[0, 1:, :] = in_ref[0] + pe_ref[...]


def kernel(inputs, class_embed, position_table):
    B, S, D = inputs.shape
    pe0 = position_table[0:1]
    pe = position_table[1:S + 1]
    return pl.pallas_call(
        _body,
        grid=(B,),
        in_specs=[
            pl.BlockSpec((1, S, D), lambda b: (b, 0, 0)),
            pl.BlockSpec((1, 1, D), lambda b: (0, 0, 0)),
            pl.BlockSpec((1, D), lambda b: (0, 0)),
            pl.BlockSpec((S, D), lambda b: (0, 0)),
        ],
        out_specs=pl.BlockSpec((1, S + 1, D), lambda b: (b, 0, 0)),
        out_shape=jax.ShapeDtypeStruct((B, S + 1, D), jnp.float32),
    )(inputs, class_embed, pe0, pe)


# speedup vs baseline: 1.0764x; 1.0679x over previous
import jax
import jax.numpy as jnp
from jax.experimental import pallas as pl


_BB = 2


def _body(in_ref, ce_ref, pe0_ref, pe_ref, out_ref):
    row0 = ce_ref[0] + pe0_ref[...]
    for j in range(_BB):
        out_ref[j, 0:1, :] = row0
        out_ref[j, 1:, :] = in_ref[j] + pe_ref[...]


def kernel(inputs, class_embed, position_table):
    B, S, D = inputs.shape
    pe0 = position_table[0:1]
    pe = position_table[1:S + 1]
    return pl.pallas_call(
        _body,
        grid=(B // _BB,),
        in_specs=[
            pl.BlockSpec((_BB, S, D), lambda b: (b, 0, 0)),
            pl.BlockSpec((1, 1, D), lambda b: (0, 0, 0)),
            pl.BlockSpec((1, D), lambda b: (0, 0)),
            pl.BlockSpec((S, D), lambda b: (0, 0)),
        ],
        out_specs=pl.BlockSpec((_BB, S + 1, D), lambda b: (b, 0, 0)),
        out_shape=jax.ShapeDtypeStruct((B, S + 1, D), jnp.float32),
    )(inputs, class_embed, pe0, pe)


# speedup vs baseline: 1.0897x; 1.0123x over previous
import jax
import jax.numpy as jnp
from jax.experimental import pallas as pl


_BB = 4


def _body(in_ref, ce_ref, pe0_ref, pe_ref, out_ref):
    row0 = ce_ref[0] + pe0_ref[...]
    for j in range(_BB):
        out_ref[j, 0:1, :] = row0
        out_ref[j, 1:, :] = in_ref[j] + pe_ref[...]


def kernel(inputs, class_embed, position_table):
    B, S, D = inputs.shape
    pe0 = position_table[0:1]
    pe = position_table[1:S + 1]
    return pl.pallas_call(
        _body,
        grid=(B // _BB,),
        in_specs=[
            pl.BlockSpec((_BB, S, D), lambda b: (b, 0, 0)),
            pl.BlockSpec((1, 1, D), lambda b: (0, 0, 0)),
            pl.BlockSpec((1, D), lambda b: (0, 0)),
            pl.BlockSpec((S, D), lambda b: (0, 0)),
        ],
        out_specs=pl.BlockSpec((_BB, S + 1, D), lambda b: (b, 0, 0)),
        out_shape=jax.ShapeDtypeStruct((B, S + 1, D), jnp.float32),
    )(inputs, class_embed, pe0, pe)


# speedup vs baseline: 1.1041x; 1.0132x over previous
import jax
import jax.numpy as jnp
from jax.experimental import pallas as pl


_BB = 8


def _body(in_ref, ce_ref, pe0_ref, pe_ref, out_ref):
    row0 = ce_ref[0] + pe0_ref[...]
    for j in range(_BB):
        out_ref[j, 0:1, :] = row0
        out_ref[j, 1:, :] = in_ref[j] + pe_ref[...]


def kernel(inputs, class_embed, position_table):
    B, S, D = inputs.shape
    pe0 = position_table[0:1]
    pe = position_table[1:S + 1]
    return pl.pallas_call(
        _body,
        grid=(B // _BB,),
        in_specs=[
            pl.BlockSpec((_BB, S, D), lambda b: (b, 0, 0)),
            pl.BlockSpec((1, 1, D), lambda b: (0, 0, 0)),
            pl.BlockSpec((1, D), lambda b: (0, 0)),
            pl.BlockSpec((S, D), lambda b: (0, 0)),
        ],
        out_specs=pl.BlockSpec((_BB, S + 1, D), lambda b: (b, 0, 0)),
        out_shape=jax.ShapeDtypeStruct((B, S + 1, D), jnp.float32),
    )(inputs, class_embed, pe0, pe)
